# baseline (speedup 1.0000x reference)
import jax
import jax.numpy as jnp
from jax import lax
from jax.experimental import pallas as pl
from jax.experimental.pallas import tpu as pltpu
from jax.experimental.pallas import tpu_sc as plsc

N_USERS = 1000000
N_ITEMS = 1000000
EMBED_DIM = 16
BATCH = 16384
LAM = 1.0

NC = 2
NS = 16
L = 16
NW = NC * NS
BPW = BATCH // NW
NCHUNK = 4
CHUNK = BPW // NCHUNK
GPC = CHUNK // L

_LN2 = 0.69314718
_SQRT2 = 1.4142135


def _ln(x):
    xi = lax.bitcast_convert_type(x, jnp.int32)
    e = lax.shift_right_arithmetic(xi, 23) - 127
    mi = jnp.bitwise_or(jnp.bitwise_and(xi, 0x007FFFFF), 0x3F800000)
    m = lax.bitcast_convert_type(mi, jnp.float32)
    big = m > _SQRT2
    m = jnp.where(big, m * 0.5, m)
    e = jnp.where(big, e + 1, e)
    ef = e.astype(jnp.float32)
    s = (m - 1.0) / (m + 1.0)
    z = s * s
    p = 2.0 * s * (1.0 + z * (1.0 / 3.0 + z * (0.2 + z * (1.0 / 7.0 + z * (1.0 / 9.0)))))
    return ef * _LN2 + p


def _sigmoid(g):
    return 1.0 / (1.0 + jnp.exp(-g))


def _sc_body(users_hbm, pos_hbm, neg_hbm, utab_hbm, itab_hbm,
             mu_hbm, mi_hbm,
             pos_out, neg_out, reg_out,
             u_idx, p_idx, n_idx,
             ub_idx, pb_idx, nb_idx,
             um_idx, pm_idx, nm_idx,
             u_blk, p_blk, n_blk,
             mu_rows, mp_rows, mn_rows,
             pos_v, neg_v, reg_v, sem_m, sem_e):
    wid = lax.axis_index("s") * NC + lax.axis_index("c")
    base = wid * BPW

    for j in range(NCHUNK):
        off = base + j * CHUNK
        pltpu.sync_copy(users_hbm.at[pl.ds(off, CHUNK)], u_idx.at[j])
        pltpu.sync_copy(pos_hbm.at[pl.ds(off, CHUNK)], p_idx.at[j])
        pltpu.sync_copy(neg_hbm.at[pl.ds(off, CHUNK)], n_idx.at[j])

    for j in range(NCHUNK):
        for r in range(GPC):
            sl = pl.ds(r * L, L)
            iu = u_idx[j, sl]
            ip = p_idx[j, sl]
            in_ = n_idx[j, sl]
            ub_idx[j, sl] = lax.shift_right_logical(iu, 3)
            pb_idx[j, sl] = lax.shift_right_logical(ip, 3)
            nb_idx[j, sl] = lax.shift_right_logical(in_, 3)
            um_idx[j, sl] = lax.shift_right_logical(iu, 4)
            pm_idx[j, sl] = lax.shift_right_logical(ip, 4)
            nm_idx[j, sl] = lax.shift_right_logical(in_, 4)

    mass_copies = []
    for j in range(NCHUNK):
        mass_copies.append(
            pltpu.async_copy(mu_hbm.at[um_idx.at[j]], mu_rows.at[j], sem_m))
        mass_copies.append(
            pltpu.async_copy(mi_hbm.at[pm_idx.at[j]], mp_rows.at[j], sem_m))
        mass_copies.append(
            pltpu.async_copy(mi_hbm.at[nm_idx.at[j]], mn_rows.at[j], sem_m))
    for c in mass_copies:
        c.wait()

    iota = lax.iota(jnp.int32, L)
    racc = jnp.zeros((L,), jnp.float32)

    for j in range(NCHUNK):
        cu = pltpu.async_copy(utab_hbm.at[ub_idx.at[j]], u_blk, sem_e)
        cp = pltpu.async_copy(itab_hbm.at[pb_idx.at[j]], p_blk, sem_e)
        cn = pltpu.async_copy(itab_hbm.at[nb_idx.at[j]], n_blk, sem_e)
        cu.wait()
        cp.wait()
        cn.wait()

        mu_j = mu_rows.at[j]
        mp_j = mp_rows.at[j]
        mn_j = mn_rows.at[j]
        u_idx_j = u_idx.at[j]
        p_idx_j = p_idx.at[j]
        n_idx_j = n_idx.at[j]

        def group(gg, racc, j=j, mu_j=mu_j, mp_j=mp_j, mn_j=mn_j,
                  u_idx_j=u_idx_j, p_idx_j=p_idx_j, n_idx_j=n_idx_j):
            rr = gg * L + iota
            iu = plsc.load_gather(u_idx_j, [rr])
            ip = plsc.load_gather(p_idx_j, [rr])
            in_ = plsc.load_gather(n_idx_j, [rr])
            cu_ = jnp.bitwise_and(iu, 7) * L
            cp_ = jnp.bitwise_and(ip, 7) * L
            cn_ = jnp.bitwise_and(in_, 7) * L

            accp = jnp.zeros((L,), jnp.float32)
            accn = jnp.zeros((L,), jnp.float32)
            for d in range(EMBED_DIM):
                u = plsc.load_gather(u_blk, [rr, cu_ + d])
                p = plsc.load_gather(p_blk, [rr, cp_ + d])
                n = plsc.load_gather(n_blk, [rr, cn_ + d])
                dp = u - p
                dn = u - n
                accp = accp + dp * dp
                accn = accn + dn * dn

            mu = plsc.load_gather(mu_j, [rr, jnp.bitwise_and(iu, 15)])
            mp = plsc.load_gather(mp_j, [rr, jnp.bitwise_and(ip, 15)])
            mn = plsc.load_gather(mn_j, [rr, jnp.bitwise_and(in_, 15)])

            lmu = _ln(jnp.maximum(mu, 0.0) + 1.0)
            lmp = _ln(jnp.maximum(mp, 0.0) + 1.0)
            lmn = _ln(jnp.maximum(mn, 0.0) + 1.0)
            dpos = LAM * _ln(accp + 0.01)
            dneg = LAM * _ln(accn + 0.01)

            sp = _sigmoid(lmu * lmp - dpos)
            sn = _sigmoid(lmu * lmn - dneg)

            pos_v[pl.ds(j * CHUNK + gg * L, L)] = sp
            neg_v[pl.ds(j * CHUNK + gg * L, L)] = sn

            return racc + mu * mu + mp * mp + mn * mn * (1.0 / BATCH)

        racc = lax.fori_loop(0, GPC, group, racc)

    reg_v[...] = jnp.zeros((L,), jnp.float32) + jnp.sum(racc)

    pltpu.sync_copy(pos_v, pos_out.at[pl.ds(base, BPW)])
    pltpu.sync_copy(neg_v, neg_out.at[pl.ds(base, BPW)])
    pltpu.sync_copy(reg_v, reg_out.at[wid])


@jax.jit
def _run(users, pos, neg, user_table, item_table, mass_u, mass_i):
    mesh = plsc.VectorSubcoreMesh(core_axis_name="c", subcore_axis_name="s")
    k = pl.kernel(
        _sc_body,
        out_type=[
            jax.ShapeDtypeStruct((BATCH,), jnp.float32),
            jax.ShapeDtypeStruct((BATCH,), jnp.float32),
            jax.ShapeDtypeStruct((NW, L), jnp.float32),
        ],
        mesh=mesh,
        compiler_params=pltpu.CompilerParams(
            needs_layout_passes=False, use_tc_tiling_on_sc=False),
        scratch_types=[
            pltpu.VMEM((NCHUNK, CHUNK), jnp.int32),
            pltpu.VMEM((NCHUNK, CHUNK), jnp.int32),
            pltpu.VMEM((NCHUNK, CHUNK), jnp.int32),
            pltpu.VMEM((NCHUNK, CHUNK), jnp.int32),
            pltpu.VMEM((NCHUNK, CHUNK), jnp.int32),
            pltpu.VMEM((NCHUNK, CHUNK), jnp.int32),
            pltpu.VMEM((NCHUNK, CHUNK), jnp.int32),
            pltpu.VMEM((NCHUNK, CHUNK), jnp.int32),
            pltpu.VMEM((NCHUNK, CHUNK), jnp.int32),
            pltpu.VMEM((CHUNK, 128), jnp.float32),
            pltpu.VMEM((CHUNK, 128), jnp.float32),
            pltpu.VMEM((CHUNK, 128), jnp.float32),
            pltpu.VMEM((NCHUNK, CHUNK, L), jnp.float32),
            pltpu.VMEM((NCHUNK, CHUNK, L), jnp.float32),
            pltpu.VMEM((NCHUNK, CHUNK, L), jnp.float32),
            pltpu.VMEM((BPW,), jnp.float32),
            pltpu.VMEM((BPW,), jnp.float32),
            pltpu.VMEM((L,), jnp.float32),
            pltpu.SemaphoreType.DMA,
            pltpu.SemaphoreType.DMA,
        ],
    )
    ut128 = user_table.reshape(N_USERS // 8, 128)
    it128 = item_table.reshape(N_ITEMS // 8, 128)
    mu2 = mass_u.reshape(N_USERS // L, L)
    mi2 = mass_i.reshape(N_ITEMS // L, L)
    pos_s, neg_s, regp = k(users, pos, neg, ut128, it128, mu2, mi2)
    reg_loss = 0.5 * jnp.sum(regp[:, 0])
    return pos_s.reshape(BATCH, 1), neg_s.reshape(BATCH, 1), reg_loss


def kernel(users, pos, neg, user_table, item_table, mass_u, mass_i):
    return _run(users.astype(jnp.int32), pos.astype(jnp.int32),
                neg.astype(jnp.int32), user_table, item_table,
                mass_u, mass_i)
